# baseline (speedup 1.0000x reference)
import functools

import jax
import jax.numpy as jnp
from jax import lax
from jax.experimental import pallas as pl
from jax.experimental.pallas import tpu as pltpu
from jax.experimental.pallas import tpu_sc as plsc

_N = 10000
_E = 320000
_D = 128
_H = 64

_C = 128
_NCH = _E // _C
_NW = 32
_CH_PER_W = _NCH // _NW
_NPAIR = (_CH_PER_W + 2) // 2
_NPAD = 10112

_MESH = plsc.VectorSubcoreMesh(core_axis_name="c", subcore_axis_name="s")


def _edge_mm_body(b_ref, w_ref, bias_ref, o_ref):
    o_ref[...] = (
        jnp.dot(b_ref[...], w_ref[...], preferred_element_type=jnp.float32)
        + bias_ref[...]
    )


def _edge_mm(bond_feats, w_edge_t, b_edge):
    be = 8000
    return pl.pallas_call(
        _edge_mm_body,
        grid=(_E // be,),
        in_specs=[
            pl.BlockSpec((be, _H), lambda i: (i, 0)),
            pl.BlockSpec((_H, _D), lambda i: (0, 0)),
            pl.BlockSpec((1, _D), lambda i: (0, 0)),
        ],
        out_specs=pl.BlockSpec((be, _D), lambda i: (i, 0)),
        out_shape=jax.ShapeDtypeStruct((_E, _D), jnp.float32),
    )(bond_feats, w_edge_t, b_edge.reshape(1, _D))


@functools.partial(
    pl.kernel,
    out_type=jax.ShapeDtypeStruct((2, _NPAD, _D), jnp.float32),
    mesh=_MESH,
    scratch_types=[
        pltpu.VMEM((2, _C), jnp.int32),
        pltpu.VMEM((2, _C), jnp.int32),
        pltpu.VMEM((2, _C, _D), jnp.float32),
        pltpu.VMEM((_C, _D), jnp.float32),
        pltpu.VMEM_SHARED((_NPAD, _D), jnp.float32),
        pltpu.SemaphoreType.DMA,
        pltpu.SemaphoreType.DMA,
        pltpu.SemaphoreType.DMA,
        pltpu.SemaphoreType.DMA,
    ],
)
def _sc_seg_sum(node_hbm, e_hbm, src_hbm, dst_hbm, out,
                src_v, dst_v, rows_v, erows_v, acc_s,
                gsem0, gsem1, ssem0, ssem1):
    gsems = (gsem0, gsem1)
    ssems = (ssem0, ssem1)
    c = lax.axis_index("c")
    t = lax.axis_index("s")
    wid = t * 2 + c
    ng = jnp.where(wid < _NCH % _NW, _CH_PER_W + 1, _CH_PER_W)

    zero16 = jnp.zeros((16,), jnp.float32)

    def zrow(i, carry):
        for j in range(_D // 16):
            rows_v[0, i, pl.ds(j * 16, 16)] = zero16
        return carry
    lax.fori_loop(0, _C, zrow, 0)

    row0 = t * 632
    for q in range(4):
        pltpu.sync_copy(rows_v.at[0], acc_s.at[pl.ds(row0 + q * 128, 128)])
    pltpu.sync_copy(rows_v.at[0].at[pl.ds(0, 120)],
                    acc_s.at[pl.ds(row0 + 512, 120)])

    plsc.subcore_barrier()

    def drain_scatter(b):
        pltpu.make_async_copy(
            rows_v.at[b], acc_s.at[dst_v.at[b]], ssems[b]).wait()

    def pair(gg, carry):
        for b in (0, 1):
            g = gg * 2 + b

            @pl.when(g < ng)
            def _prefetch():
                base = (g * _NW + wid) * _C

                @pl.when(g >= 2)
                def _drain():
                    drain_scatter(b)
                pltpu.sync_copy(src_hbm.at[pl.ds(base, _C)], src_v.at[b])
                pltpu.sync_copy(dst_hbm.at[pl.ds(base, _C)], dst_v.at[b])
                pltpu.async_copy(
                    node_hbm.at[src_v.at[b]], rows_v.at[b], gsems[b])

            pb = 1 - b

            @pl.when(jnp.logical_and(g >= 1, g - 1 < ng))
            def _process():
                gp = g - 1
                base = (gp * _NW + wid) * _C
                pltpu.sync_copy(e_hbm.at[pl.ds(base, _C)], erows_v)
                pltpu.make_async_copy(
                    node_hbm.at[src_v.at[pb]], rows_v.at[pb],
                    gsems[pb]).wait()

                def addrow(i, carry2):
                    for j in range(_D // 16):
                        sl = pl.ds(j * 16, 16)
                        rows_v[pb, i, sl] = (rows_v[pb, i, sl]
                                             + erows_v[i, sl])
                    return carry2
                lax.fori_loop(0, _C, addrow, 0)

                pltpu.async_copy(
                    rows_v.at[pb], acc_s.at[dst_v.at[pb]], ssems[pb],
                    add=True)
        return carry
    lax.fori_loop(0, (_CH_PER_W + 1 + 2) // 2, pair, 0)

    for b in (0, 1):
        drain_scatter(b)

    plsc.subcore_barrier()

    for q in range(4):
        pltpu.sync_copy(acc_s.at[pl.ds(row0 + q * 128, 128)],
                        out.at[c, pl.ds(row0 + q * 128, 128)])
    pltpu.sync_copy(acc_s.at[pl.ds(row0 + 512, 120)],
                    out.at[c, pl.ds(row0 + 512, 120)])


def _final_body(agg_ref, wf_ref, bf_ref, o_ref):
    agg = agg_ref[0] + agg_ref[1]
    h = (jnp.dot(agg, wf_ref[...], preferred_element_type=jnp.float32)
         + bf_ref[...])
    o_ref[...] = h * jax.nn.sigmoid(h)


def _final_mm(aggp, w_fc_t, b_fc):
    bn = 2000
    return pl.pallas_call(
        _final_body,
        grid=(_N // bn,),
        in_specs=[
            pl.BlockSpec((2, bn, _D), lambda i: (0, i, 0)),
            pl.BlockSpec((_D, _D), lambda i: (0, 0)),
            pl.BlockSpec((1, _D), lambda i: (0, 0)),
        ],
        out_specs=pl.BlockSpec((bn, _D), lambda i: (i, 0)),
        out_shape=jax.ShapeDtypeStruct((_N, _D), jnp.float32),
    )(aggp, w_fc_t, b_fc.reshape(1, _D))


def kernel(node_feats, edge_index, bond_feats, W_edge, b_edge, W_fc, b_fc):
    src = edge_index[0]
    dst = edge_index[1]
    e = _edge_mm(bond_feats, W_edge.T, b_edge)
    aggp = _sc_seg_sum(node_feats, e, src, dst)
    out = _final_mm(aggp, W_fc.T, b_fc)
    return (out, e)


# speedup vs baseline: 1.0305x; 1.0305x over previous
import functools

import jax
import jax.numpy as jnp
from jax import lax
from jax.experimental import pallas as pl
from jax.experimental.pallas import tpu as pltpu
from jax.experimental.pallas import tpu_sc as plsc

_N = 10000
_E = 320000
_D = 128
_H = 64

_C = 128
_NCH = _E // _C
_NW = 32
_CH_PER_W = _NCH // _NW
_NPAIR = (_CH_PER_W + 2) // 2
_NPAD = 10240

_MESH = plsc.VectorSubcoreMesh(core_axis_name="c", subcore_axis_name="s")


def _edge_mm_body(b_ref, w_ref, bias_ref, o_ref):
    o_ref[...] = (
        jnp.dot(b_ref[...], w_ref[...], preferred_element_type=jnp.float32)
        + bias_ref[...]
    )


def _edge_mm(bond_feats, w_edge_t, b_edge):
    be = 8000
    return pl.pallas_call(
        _edge_mm_body,
        grid=(_E // be,),
        in_specs=[
            pl.BlockSpec((be, _H), lambda i: (i, 0)),
            pl.BlockSpec((_H, _D), lambda i: (0, 0)),
            pl.BlockSpec((1, _D), lambda i: (0, 0)),
        ],
        out_specs=pl.BlockSpec((be, _D), lambda i: (i, 0)),
        out_shape=jax.ShapeDtypeStruct((_E, _D), jnp.float32),
    )(bond_feats, w_edge_t, b_edge.reshape(1, _D))


def _zero_rows(rows_ref):
    zero16 = jnp.zeros((16,), jnp.float32)

    def zrow(i, carry):
        for j in range(_D // 16):
            rows_ref[i, pl.ds(j * 16, 16)] = zero16
        return carry
    lax.fori_loop(0, _C, zrow, 0)


def _prologue(t, rows_v, acc_s):
    _zero_rows(rows_v.at[0])
    row0 = t * 640
    for q in range(5):
        pltpu.sync_copy(rows_v.at[0], acc_s.at[pl.ds(row0 + q * 128, 128)])
    plsc.subcore_barrier()
    return row0


def _epilogue(c, row0, rows_v, dst_v, acc_s, ssems, out):
    for b in (0, 1):
        pltpu.make_async_copy(
            rows_v.at[b], acc_s.at[dst_v.at[b]], ssems[b]).wait()
    plsc.subcore_barrier()
    for q in range(5):
        pltpu.sync_copy(acc_s.at[pl.ds(row0 + q * 128, 128)],
                        out.at[c, pl.ds(row0 + q * 128, 128)])


@functools.partial(
    pl.kernel,
    out_type=jax.ShapeDtypeStruct((2, _NPAD, _D), jnp.float32),
    mesh=_MESH,
    scratch_types=[
        pltpu.VMEM((2, _C), jnp.int32),
        pltpu.VMEM((2, _C), jnp.int32),
        pltpu.VMEM((2, _C, _D), jnp.float32),
        pltpu.VMEM_SHARED((_NPAD, _D), jnp.float32),
        pltpu.SemaphoreType.DMA,
        pltpu.SemaphoreType.DMA,
        pltpu.SemaphoreType.DMA,
    ],
)
def _sc_node_seg_sum(node_hbm, src_hbm, dst_hbm, out,
                     src_v, dst_v, rows_v, acc_s, lsem, ssem0, ssem1):
    ssems = (ssem0, ssem1)
    c = lax.axis_index("c")
    t = lax.axis_index("s")
    wid = t * 2 + c
    row0 = _prologue(t, rows_v, acc_s)
    ng = jnp.where(wid < _NCH % _NW, _CH_PER_W + 1, _CH_PER_W)

    def pair(gg, carry):
        for b in (0, 1):
            g = gg * 2 + b
            base = (g * _NW + wid) * _C

            @pl.when(jnp.logical_and(g >= 2, g < ng))
            def _drain():
                pltpu.make_async_copy(
                    rows_v.at[b], acc_s.at[dst_v.at[b]], ssems[b]).wait()

            @pl.when(g < ng)
            def _work():
                pltpu.sync_copy(dst_hbm.at[pl.ds(base, _C)], dst_v.at[b])
                pltpu.sync_copy(src_hbm.at[pl.ds(base, _C)], src_v.at[b])
                pltpu.async_copy(
                    node_hbm.at[src_v.at[b]], rows_v.at[b], lsem).wait()
                pltpu.async_copy(
                    rows_v.at[b], acc_s.at[dst_v.at[b]], ssems[b], add=True)
        return carry
    lax.fori_loop(0, _NPAIR, pair, 0)

    _epilogue(c, row0, rows_v, dst_v, acc_s, ssems, out)


@functools.partial(
    pl.kernel,
    out_type=jax.ShapeDtypeStruct((2, _NPAD, _D), jnp.float32),
    mesh=_MESH,
    scratch_types=[
        pltpu.VMEM((2, _C), jnp.int32),
        pltpu.VMEM((2, _C, _D), jnp.float32),
        pltpu.VMEM_SHARED((_NPAD, _D), jnp.float32),
        pltpu.SemaphoreType.DMA,
        pltpu.SemaphoreType.DMA,
        pltpu.SemaphoreType.DMA,
    ],
)
def _sc_e_seg_sum(e_hbm, dst_hbm, out,
                  dst_v, rows_v, acc_s, lsem, ssem0, ssem1):
    ssems = (ssem0, ssem1)
    c = lax.axis_index("c")
    t = lax.axis_index("s")
    wid = t * 2 + c
    row0 = _prologue(t, rows_v, acc_s)
    ng = jnp.where(wid < _NCH % _NW, _CH_PER_W + 1, _CH_PER_W)

    def pair(gg, carry):
        for b in (0, 1):
            g = gg * 2 + b
            base = (g * _NW + wid) * _C

            @pl.when(jnp.logical_and(g >= 2, g < ng))
            def _drain():
                pltpu.make_async_copy(
                    rows_v.at[b], acc_s.at[dst_v.at[b]], ssems[b]).wait()

            @pl.when(g < ng)
            def _work():
                pltpu.sync_copy(dst_hbm.at[pl.ds(base, _C)], dst_v.at[b])
                pltpu.async_copy(
                    e_hbm.at[pl.ds(base, _C)], rows_v.at[b], lsem).wait()
                pltpu.async_copy(
                    rows_v.at[b], acc_s.at[dst_v.at[b]], ssems[b], add=True)
        return carry
    lax.fori_loop(0, _NPAIR, pair, 0)

    _epilogue(c, row0, rows_v, dst_v, acc_s, ssems, out)


def _final_body(aggn_ref, agge_ref, wf_ref, bf_ref, o_ref):
    agg = (aggn_ref[0] + aggn_ref[1]) + (agge_ref[0] + agge_ref[1])
    h = (jnp.dot(agg, wf_ref[...], preferred_element_type=jnp.float32)
         + bf_ref[...])
    o_ref[...] = h * jax.nn.sigmoid(h)


def _final_mm(aggn, agge, w_fc_t, b_fc):
    bn = 2000
    return pl.pallas_call(
        _final_body,
        grid=(_N // bn,),
        in_specs=[
            pl.BlockSpec((2, bn, _D), lambda i: (0, i, 0)),
            pl.BlockSpec((2, bn, _D), lambda i: (0, i, 0)),
            pl.BlockSpec((_D, _D), lambda i: (0, 0)),
            pl.BlockSpec((1, _D), lambda i: (0, 0)),
        ],
        out_specs=pl.BlockSpec((bn, _D), lambda i: (i, 0)),
        out_shape=jax.ShapeDtypeStruct((_N, _D), jnp.float32),
    )(aggn, agge, w_fc_t, b_fc.reshape(1, _D))


def kernel(node_feats, edge_index, bond_feats, W_edge, b_edge, W_fc, b_fc):
    src = edge_index[0]
    dst = edge_index[1]
    aggn = _sc_node_seg_sum(node_feats, src, dst)
    e = _edge_mm(bond_feats, W_edge.T, b_edge)
    agge = _sc_e_seg_sum(e, dst)
    out = _final_mm(aggn, agge, W_fc.T, b_fc)
    return (out, e)


# speedup vs baseline: 1.0385x; 1.0078x over previous
import functools

import jax
import jax.numpy as jnp
from jax import lax
from jax.experimental import pallas as pl
from jax.experimental.pallas import tpu as pltpu
from jax.experimental.pallas import tpu_sc as plsc

_N = 10000
_E = 320000
_D = 128
_H = 64

_C = 128
_NCH = _E // _C
_NW = 32
_CH_PER_W = _NCH // _NW
_NPAIR = (_CH_PER_W + 2) // 2
_NPAD = 10240

_MESH = plsc.VectorSubcoreMesh(core_axis_name="c", subcore_axis_name="s")


def _edge_mm_body(b_ref, w_ref, bias_ref, o_ref):
    o_ref[...] = (
        jnp.dot(b_ref[...], w_ref[...], preferred_element_type=jnp.float32)
        + bias_ref[...]
    )


def _edge_mm(bond_feats, w_edge_t, b_edge):
    be = 16000
    return pl.pallas_call(
        _edge_mm_body,
        grid=(_E // be,),
        in_specs=[
            pl.BlockSpec((be, _H), lambda i: (i, 0)),
            pl.BlockSpec((_H, _D), lambda i: (0, 0)),
            pl.BlockSpec((1, _D), lambda i: (0, 0)),
        ],
        out_specs=pl.BlockSpec((be, _D), lambda i: (i, 0)),
        out_shape=jax.ShapeDtypeStruct((_E, _D), jnp.float32),
    )(bond_feats, w_edge_t, b_edge.reshape(1, _D))


def _zero_rows(rows_ref):
    zero16 = jnp.zeros((16,), jnp.float32)

    def zrow(i, carry):
        for j in range(_D // 16):
            rows_ref[i, pl.ds(j * 16, 16)] = zero16
        return carry
    lax.fori_loop(0, _C, zrow, 0)


def _prologue(t, rows_v, acc_s):
    _zero_rows(rows_v.at[0])
    row0 = t * 640
    for q in range(5):
        pltpu.sync_copy(rows_v.at[0], acc_s.at[pl.ds(row0 + q * 128, 128)])
    plsc.subcore_barrier()
    return row0


def _epilogue(c, row0, rows_v, dst_v, acc_s, ssems, out):
    for b in (0, 1):
        pltpu.make_async_copy(
            rows_v.at[b], acc_s.at[dst_v.at[b]], ssems[b]).wait()
    plsc.subcore_barrier()
    for q in range(5):
        pltpu.sync_copy(acc_s.at[pl.ds(row0 + q * 128, 128)],
                        out.at[c, pl.ds(row0 + q * 128, 128)])


@functools.partial(
    pl.kernel,
    out_type=jax.ShapeDtypeStruct((2, _NPAD, _D), jnp.float32),
    mesh=_MESH,
    scratch_types=[
        pltpu.VMEM((2, _C), jnp.int32),
        pltpu.VMEM((2, _C), jnp.int32),
        pltpu.VMEM((2, _C, _D), jnp.float32),
        pltpu.VMEM_SHARED((_NPAD, _D), jnp.float32),
        pltpu.SemaphoreType.DMA,
        pltpu.SemaphoreType.DMA,
        pltpu.SemaphoreType.DMA,
    ],
)
def _sc_node_seg_sum(node_hbm, src_hbm, dst_hbm, out,
                     src_v, dst_v, rows_v, acc_s, lsem, ssem0, ssem1):
    ssems = (ssem0, ssem1)
    c = lax.axis_index("c")
    t = lax.axis_index("s")
    wid = t * 2 + c
    row0 = _prologue(t, rows_v, acc_s)
    ng = jnp.where(wid < _NCH % _NW, _CH_PER_W + 1, _CH_PER_W)

    def pair(gg, carry):
        for b in (0, 1):
            g = gg * 2 + b
            base = (g * _NW + wid) * _C

            @pl.when(jnp.logical_and(g >= 2, g < ng))
            def _drain():
                pltpu.make_async_copy(
                    rows_v.at[b], acc_s.at[dst_v.at[b]], ssems[b]).wait()

            @pl.when(g < ng)
            def _work():
                pltpu.sync_copy(dst_hbm.at[pl.ds(base, _C)], dst_v.at[b])
                pltpu.sync_copy(src_hbm.at[pl.ds(base, _C)], src_v.at[b])
                pltpu.async_copy(
                    node_hbm.at[src_v.at[b]], rows_v.at[b], lsem).wait()
                pltpu.async_copy(
                    rows_v.at[b], acc_s.at[dst_v.at[b]], ssems[b], add=True)
        return carry
    lax.fori_loop(0, _NPAIR, pair, 0)

    _epilogue(c, row0, rows_v, dst_v, acc_s, ssems, out)


@functools.partial(
    pl.kernel,
    out_type=jax.ShapeDtypeStruct((2, _NPAD, _D), jnp.float32),
    mesh=_MESH,
    scratch_types=[
        pltpu.VMEM((2, _C), jnp.int32),
        pltpu.VMEM((2, _C, _D), jnp.float32),
        pltpu.VMEM_SHARED((_NPAD, _D), jnp.float32),
        pltpu.SemaphoreType.DMA,
        pltpu.SemaphoreType.DMA,
        pltpu.SemaphoreType.DMA,
    ],
)
def _sc_e_seg_sum(e_hbm, dst_hbm, out,
                  dst_v, rows_v, acc_s, lsem, ssem0, ssem1):
    ssems = (ssem0, ssem1)
    c = lax.axis_index("c")
    t = lax.axis_index("s")
    wid = t * 2 + c
    row0 = _prologue(t, rows_v, acc_s)
    ng = jnp.where(wid < _NCH % _NW, _CH_PER_W + 1, _CH_PER_W)

    def pair(gg, carry):
        for b in (0, 1):
            g = gg * 2 + b
            base = (g * _NW + wid) * _C

            @pl.when(jnp.logical_and(g >= 2, g < ng))
            def _drain():
                pltpu.make_async_copy(
                    rows_v.at[b], acc_s.at[dst_v.at[b]], ssems[b]).wait()

            @pl.when(g < ng)
            def _work():
                pltpu.sync_copy(dst_hbm.at[pl.ds(base, _C)], dst_v.at[b])
                pltpu.async_copy(
                    e_hbm.at[pl.ds(base, _C)], rows_v.at[b], lsem).wait()
                pltpu.async_copy(
                    rows_v.at[b], acc_s.at[dst_v.at[b]], ssems[b], add=True)
        return carry
    lax.fori_loop(0, _NPAIR, pair, 0)

    _epilogue(c, row0, rows_v, dst_v, acc_s, ssems, out)


def _final_body(aggn_ref, agge_ref, wf_ref, bf_ref, o_ref):
    agg = (aggn_ref[0] + aggn_ref[1]) + (agge_ref[0] + agge_ref[1])
    h = (jnp.dot(agg, wf_ref[...], preferred_element_type=jnp.float32)
         + bf_ref[...])
    o_ref[...] = h * jax.nn.sigmoid(h)


def _final_mm(aggn, agge, w_fc_t, b_fc):
    bn = 2000
    return pl.pallas_call(
        _final_body,
        grid=(_N // bn,),
        in_specs=[
            pl.BlockSpec((2, bn, _D), lambda i: (0, i, 0)),
            pl.BlockSpec((2, bn, _D), lambda i: (0, i, 0)),
            pl.BlockSpec((_D, _D), lambda i: (0, 0)),
            pl.BlockSpec((1, _D), lambda i: (0, 0)),
        ],
        out_specs=pl.BlockSpec((bn, _D), lambda i: (i, 0)),
        out_shape=jax.ShapeDtypeStruct((_N, _D), jnp.float32),
    )(aggn, agge, w_fc_t, b_fc.reshape(1, _D))


def kernel(node_feats, edge_index, bond_feats, W_edge, b_edge, W_fc, b_fc):
    src = edge_index[0]
    dst = edge_index[1]
    aggn = _sc_node_seg_sum(node_feats, src, dst)
    e = _edge_mm(bond_feats, W_edge.T, b_edge)
    agge = _sc_e_seg_sum(e, dst)
    out = _final_mm(aggn, agge, W_fc.T, b_fc)
    return (out, e)


# speedup vs baseline: 1.2144x; 1.1693x over previous
import functools

import jax
import jax.numpy as jnp
from jax import lax
from jax.experimental import pallas as pl
from jax.experimental.pallas import tpu as pltpu
from jax.experimental.pallas import tpu_sc as plsc

_N = 10000
_E = 320000
_D = 128
_H = 64

_C = 128
_NCH = _E // _C
_NW = 32
_CH_PER_W = _NCH // _NW
_NPAIR = (_CH_PER_W + 2) // 2
_NPAD = 10240

_MESH = plsc.VectorSubcoreMesh(core_axis_name="c", subcore_axis_name="s")


def _edge_mm_body(b_ref, w_ref, bias_ref, o_ref):
    o_ref[...] = (
        jnp.dot(b_ref[...], w_ref[...], preferred_element_type=jnp.float32)
        + bias_ref[...]
    )


def _edge_mm(bond_feats, w_edge_t, b_edge):
    be = 16000
    return pl.pallas_call(
        _edge_mm_body,
        grid=(_E // be,),
        in_specs=[
            pl.BlockSpec((be, _H), lambda i: (i, 0)),
            pl.BlockSpec((_H, _D), lambda i: (0, 0)),
            pl.BlockSpec((1, _D), lambda i: (0, 0)),
        ],
        out_specs=pl.BlockSpec((be, _D), lambda i: (i, 0)),
        out_shape=jax.ShapeDtypeStruct((_E, _D), jnp.float32),
    )(bond_feats, w_edge_t, b_edge.reshape(1, _D))


def _zero_rows(rows_ref):
    zero16 = jnp.zeros((16,), jnp.float32)

    def zrow(i, carry):
        for j in range(_D // 16):
            rows_ref[i, pl.ds(j * 16, 16)] = zero16
        return carry
    lax.fori_loop(0, _C, zrow, 0)


def _prologue(t, rows_v, acc_s):
    _zero_rows(rows_v.at[0])
    row0 = t * 640
    for q in range(5):
        pltpu.sync_copy(rows_v.at[0], acc_s.at[pl.ds(row0 + q * 128, 128)])
    plsc.subcore_barrier()
    return row0


def _epilogue(c, row0, rows_v, dst_v, acc_s, ssems, out):
    for b in (0, 1):
        pltpu.make_async_copy(
            rows_v.at[b], acc_s.at[dst_v.at[b]], ssems[b]).wait()
    plsc.subcore_barrier()
    for q in range(5):
        pltpu.sync_copy(acc_s.at[pl.ds(row0 + q * 128, 128)],
                        out.at[c, pl.ds(row0 + q * 128, 128)])


@functools.partial(
    pl.kernel,
    out_type=jax.ShapeDtypeStruct((2, _NPAD, _D), jnp.float32),
    mesh=_MESH,
    scratch_types=[
        pltpu.VMEM((2, _C), jnp.int32),
        pltpu.VMEM((2, _C), jnp.int32),
        pltpu.VMEM((2, _C, _D), jnp.float32),
        pltpu.VMEM_SHARED((_NPAD, _D), jnp.float32),
        pltpu.SemaphoreType.DMA,
        pltpu.SemaphoreType.DMA,
        pltpu.SemaphoreType.DMA,
        pltpu.SemaphoreType.DMA,
    ],
)
def _sc_node_seg_sum(node_hbm, src_hbm, dst_hbm, out,
                     src_v, dst_v, rows_v, acc_s,
                     gsem0, gsem1, ssem0, ssem1):
    gsems = (gsem0, gsem1)
    ssems = (ssem0, ssem1)
    c = lax.axis_index("c")
    t = lax.axis_index("s")
    wid = t * 2 + c
    row0 = _prologue(t, rows_v, acc_s)
    ng = jnp.where(wid < _NCH % _NW, _CH_PER_W + 1, _CH_PER_W)

    def pair(gg, carry):
        for b in (0, 1):
            g = gg * 2 + b
            pb = 1 - b

            @pl.when(g < ng)
            def _prefetch():
                base = (g * _NW + wid) * _C

                @pl.when(g >= 2)
                def _drain():
                    pltpu.make_async_copy(
                        rows_v.at[b], acc_s.at[dst_v.at[b]],
                        ssems[b]).wait()
                pltpu.sync_copy(dst_hbm.at[pl.ds(base, _C)], dst_v.at[b])
                pltpu.sync_copy(src_hbm.at[pl.ds(base, _C)], src_v.at[b])
                pltpu.async_copy(
                    node_hbm.at[src_v.at[b]], rows_v.at[b], gsems[b])

            @pl.when(jnp.logical_and(g >= 1, g - 1 < ng))
            def _process():
                pltpu.make_async_copy(
                    node_hbm.at[src_v.at[pb]], rows_v.at[pb],
                    gsems[pb]).wait()
                pltpu.async_copy(
                    rows_v.at[pb], acc_s.at[dst_v.at[pb]], ssems[pb],
                    add=True)
        return carry
    lax.fori_loop(0, (_CH_PER_W + 1 + 2) // 2, pair, 0)

    _epilogue(c, row0, rows_v, dst_v, acc_s, ssems, out)


@functools.partial(
    pl.kernel,
    out_type=jax.ShapeDtypeStruct((2, _NPAD, _D), jnp.float32),
    mesh=_MESH,
    scratch_types=[
        pltpu.VMEM((2, _C), jnp.int32),
        pltpu.VMEM((2, _C, _D), jnp.float32),
        pltpu.VMEM_SHARED((_NPAD, _D), jnp.float32),
        pltpu.SemaphoreType.DMA,
        pltpu.SemaphoreType.DMA,
        pltpu.SemaphoreType.DMA,
        pltpu.SemaphoreType.DMA,
    ],
)
def _sc_e_seg_sum(e_hbm, dst_hbm, out,
                  dst_v, rows_v, acc_s, lsem0, lsem1, ssem0, ssem1):
    lsems = (lsem0, lsem1)
    ssems = (ssem0, ssem1)
    c = lax.axis_index("c")
    t = lax.axis_index("s")
    wid = t * 2 + c
    row0 = _prologue(t, rows_v, acc_s)
    ng = jnp.where(wid < _NCH % _NW, _CH_PER_W + 1, _CH_PER_W)

    def pair(gg, carry):
        for b in (0, 1):
            g = gg * 2 + b
            pb = 1 - b

            @pl.when(g < ng)
            def _prefetch():
                base = (g * _NW + wid) * _C

                @pl.when(g >= 2)
                def _drain():
                    pltpu.make_async_copy(
                        rows_v.at[b], acc_s.at[dst_v.at[b]],
                        ssems[b]).wait()
                pltpu.sync_copy(dst_hbm.at[pl.ds(base, _C)], dst_v.at[b])
                pltpu.async_copy(
                    e_hbm.at[pl.ds(base, _C)], rows_v.at[b], lsems[b])

            @pl.when(jnp.logical_and(g >= 1, g - 1 < ng))
            def _process():
                gp = g - 1
                basep = (gp * _NW + wid) * _C
                pltpu.make_async_copy(
                    e_hbm.at[pl.ds(basep, _C)], rows_v.at[pb],
                    lsems[pb]).wait()
                pltpu.async_copy(
                    rows_v.at[pb], acc_s.at[dst_v.at[pb]], ssems[pb],
                    add=True)
        return carry
    lax.fori_loop(0, (_CH_PER_W + 1 + 2) // 2, pair, 0)

    _epilogue(c, row0, rows_v, dst_v, acc_s, ssems, out)


def _final_body(aggn_ref, agge_ref, wf_ref, bf_ref, o_ref):
    agg = (aggn_ref[0] + aggn_ref[1]) + (agge_ref[0] + agge_ref[1])
    h = (jnp.dot(agg, wf_ref[...], preferred_element_type=jnp.float32)
         + bf_ref[...])
    o_ref[...] = h * jax.nn.sigmoid(h)


def _final_mm(aggn, agge, w_fc_t, b_fc):
    bn = 2000
    return pl.pallas_call(
        _final_body,
        grid=(_N // bn,),
        in_specs=[
            pl.BlockSpec((2, bn, _D), lambda i: (0, i, 0)),
            pl.BlockSpec((2, bn, _D), lambda i: (0, i, 0)),
            pl.BlockSpec((_D, _D), lambda i: (0, 0)),
            pl.BlockSpec((1, _D), lambda i: (0, 0)),
        ],
        out_specs=pl.BlockSpec((bn, _D), lambda i: (i, 0)),
        out_shape=jax.ShapeDtypeStruct((_N, _D), jnp.float32),
    )(aggn, agge, w_fc_t, b_fc.reshape(1, _D))


def kernel(node_feats, edge_index, bond_feats, W_edge, b_edge, W_fc, b_fc):
    src = edge_index[0]
    dst = edge_index[1]
    aggn = _sc_node_seg_sum(node_feats, src, dst)
    e = _edge_mm(bond_feats, W_edge.T, b_edge)
    agge = _sc_e_seg_sum(e, dst)
    out = _final_mm(aggn, agge, W_fc.T, b_fc)
    return (out, e)


# speedup vs baseline: 1.2791x; 1.0533x over previous
import functools

import jax
import jax.numpy as jnp
from jax import lax
from jax.experimental import pallas as pl
from jax.experimental.pallas import tpu as pltpu
from jax.experimental.pallas import tpu_sc as plsc

_N = 10000
_E = 320000
_D = 128
_H = 64

_C = 128
_NCH = _E // _C
_NW = 32
_CH_PER_W = _NCH // _NW
_NPAIR = (_CH_PER_W + 2) // 2
_NPAD = 10240

_MESH = plsc.VectorSubcoreMesh(core_axis_name="c", subcore_axis_name="s")


def _edge_mm_body(b_ref, w_ref, bias_ref, o_ref):
    o_ref[...] = (
        jnp.dot(b_ref[...], w_ref[...], preferred_element_type=jnp.float32)
        + bias_ref[...]
    )


def _edge_mm(bond_feats, w_edge_t, b_edge):
    be = 16000
    return pl.pallas_call(
        _edge_mm_body,
        grid=(_E // be,),
        in_specs=[
            pl.BlockSpec((be, _H), lambda i: (i, 0)),
            pl.BlockSpec((_H, _D), lambda i: (0, 0)),
            pl.BlockSpec((1, _D), lambda i: (0, 0)),
        ],
        out_specs=pl.BlockSpec((be, _D), lambda i: (i, 0)),
        out_shape=jax.ShapeDtypeStruct((_E, _D), jnp.float32),
    )(bond_feats, w_edge_t, b_edge.reshape(1, _D))


def _zero_rows(rows_ref):
    zero16 = jnp.zeros((16,), jnp.float32)

    def zrow(i, carry):
        for j in range(_D // 16):
            rows_ref[i, pl.ds(j * 16, 16)] = zero16
        return carry
    lax.fori_loop(0, _C, zrow, 0)


def _prologue(t, rows_v, acc_s):
    _zero_rows(rows_v.at[0])
    row0 = t * 640
    for q in range(5):
        pltpu.sync_copy(rows_v.at[0], acc_s.at[pl.ds(row0 + q * 128, 128)])
    plsc.subcore_barrier()
    return row0


def _epilogue(c, row0, rows_v, dst_v, acc_s, ssems, out):
    for b in (0, 1):
        pltpu.make_async_copy(
            rows_v.at[b], acc_s.at[dst_v.at[b]], ssems[b]).wait()
    plsc.subcore_barrier()
    for q in range(5):
        pltpu.sync_copy(acc_s.at[pl.ds(row0 + q * 128, 128)],
                        out.at[c, pl.ds(row0 + q * 128, 128)])


@functools.partial(
    pl.kernel,
    out_type=jax.ShapeDtypeStruct((2, _NPAD, _D), jnp.float32),
    mesh=_MESH,
    scratch_types=[
        pltpu.VMEM((2, _C), jnp.int32),
        pltpu.VMEM((2, _C), jnp.int32),
        pltpu.VMEM((2, _C, _D), jnp.float32),
        pltpu.VMEM_SHARED((_NPAD, _D), jnp.float32),
        pltpu.SemaphoreType.DMA,
        pltpu.SemaphoreType.DMA,
        pltpu.SemaphoreType.DMA,
        pltpu.SemaphoreType.DMA,
        pltpu.SemaphoreType.DMA,
    ],
)
def _sc_node_seg_sum(node_hbm, src_hbm, dst_hbm, out,
                     src_v, dst_v, rows_v, acc_s,
                     isem, gsem0, gsem1, ssem0, ssem1):
    gsems = (gsem0, gsem1)
    ssems = (ssem0, ssem1)
    c = lax.axis_index("c")
    t = lax.axis_index("s")
    wid = t * 2 + c
    row0 = _prologue(t, rows_v, acc_s)
    ng = jnp.where(wid < _NCH % _NW, _CH_PER_W + 1, _CH_PER_W)

    def pair(gg, carry):
        for b in (0, 1):
            g = gg * 2 + b
            pb = 1 - b

            @pl.when(g < ng)
            def _prefetch():
                base = (g * _NW + wid) * _C

                @pl.when(g >= 2)
                def _drain():
                    pltpu.make_async_copy(
                        rows_v.at[b], acc_s.at[dst_v.at[b]],
                        ssems[b]).wait()
                cp_d = pltpu.async_copy(
                    dst_hbm.at[pl.ds(base, _C)], dst_v.at[b], isem)
                cp_s = pltpu.async_copy(
                    src_hbm.at[pl.ds(base, _C)], src_v.at[b], isem)
                cp_s.wait()
                cp_d.wait()
                pltpu.async_copy(
                    node_hbm.at[src_v.at[b]], rows_v.at[b], gsems[b])

            @pl.when(jnp.logical_and(g >= 1, g - 1 < ng))
            def _process():
                pltpu.make_async_copy(
                    node_hbm.at[src_v.at[pb]], rows_v.at[pb],
                    gsems[pb]).wait()
                pltpu.async_copy(
                    rows_v.at[pb], acc_s.at[dst_v.at[pb]], ssems[pb],
                    add=True)
        return carry
    lax.fori_loop(0, (_CH_PER_W + 1 + 2) // 2, pair, 0)

    _epilogue(c, row0, rows_v, dst_v, acc_s, ssems, out)


@functools.partial(
    pl.kernel,
    out_type=jax.ShapeDtypeStruct((2, _NPAD, _D), jnp.float32),
    mesh=_MESH,
    scratch_types=[
        pltpu.VMEM((2, _C), jnp.int32),
        pltpu.VMEM((2, _C, _D), jnp.float32),
        pltpu.VMEM_SHARED((_NPAD, _D), jnp.float32),
        pltpu.SemaphoreType.DMA,
        pltpu.SemaphoreType.DMA,
        pltpu.SemaphoreType.DMA,
        pltpu.SemaphoreType.DMA,
    ],
)
def _sc_e_seg_sum(e_hbm, dst_hbm, out,
                  dst_v, rows_v, acc_s, lsem0, lsem1, ssem0, ssem1):
    lsems = (lsem0, lsem1)
    ssems = (ssem0, ssem1)
    c = lax.axis_index("c")
    t = lax.axis_index("s")
    wid = t * 2 + c
    row0 = _prologue(t, rows_v, acc_s)
    ng = jnp.where(wid < _NCH % _NW, _CH_PER_W + 1, _CH_PER_W)

    def pair(gg, carry):
        for b in (0, 1):
            g = gg * 2 + b
            pb = 1 - b

            @pl.when(g < ng)
            def _prefetch():
                base = (g * _NW + wid) * _C

                @pl.when(g >= 2)
                def _drain():
                    pltpu.make_async_copy(
                        rows_v.at[b], acc_s.at[dst_v.at[b]],
                        ssems[b]).wait()
                pltpu.sync_copy(dst_hbm.at[pl.ds(base, _C)], dst_v.at[b])
                pltpu.async_copy(
                    e_hbm.at[pl.ds(base, _C)], rows_v.at[b], lsems[b])

            @pl.when(jnp.logical_and(g >= 1, g - 1 < ng))
            def _process():
                gp = g - 1
                basep = (gp * _NW + wid) * _C
                pltpu.make_async_copy(
                    e_hbm.at[pl.ds(basep, _C)], rows_v.at[pb],
                    lsems[pb]).wait()
                pltpu.async_copy(
                    rows_v.at[pb], acc_s.at[dst_v.at[pb]], ssems[pb],
                    add=True)
        return carry
    lax.fori_loop(0, (_CH_PER_W + 1 + 2) // 2, pair, 0)

    _epilogue(c, row0, rows_v, dst_v, acc_s, ssems, out)


def _final_body(aggn_ref, agge_ref, wf_ref, bf_ref, o_ref):
    agg = (aggn_ref[0] + aggn_ref[1]) + (agge_ref[0] + agge_ref[1])
    h = (jnp.dot(agg, wf_ref[...], preferred_element_type=jnp.float32)
         + bf_ref[...])
    o_ref[...] = h * jax.nn.sigmoid(h)


def _final_mm(aggn, agge, w_fc_t, b_fc):
    bn = 2000
    return pl.pallas_call(
        _final_body,
        grid=(_N // bn,),
        in_specs=[
            pl.BlockSpec((2, bn, _D), lambda i: (0, i, 0)),
            pl.BlockSpec((2, bn, _D), lambda i: (0, i, 0)),
            pl.BlockSpec((_D, _D), lambda i: (0, 0)),
            pl.BlockSpec((1, _D), lambda i: (0, 0)),
        ],
        out_specs=pl.BlockSpec((bn, _D), lambda i: (i, 0)),
        out_shape=jax.ShapeDtypeStruct((_N, _D), jnp.float32),
    )(aggn, agge, w_fc_t, b_fc.reshape(1, _D))


def kernel(node_feats, edge_index, bond_feats, W_edge, b_edge, W_fc, b_fc):
    src = edge_index[0]
    dst = edge_index[1]
    aggn = _sc_node_seg_sum(node_feats, src, dst)
    e = _edge_mm(bond_feats, W_edge.T, b_edge)
    agge = _sc_e_seg_sum(e, dst)
    out = _final_mm(aggn, agge, W_fc.T, b_fc)
    return (out, e)
